# baseline (speedup 1.0000x reference)
import jax
import jax.numpy as jnp
from jax import lax
from jax.experimental import pallas as pl
from jax.experimental.pallas import tpu as pltpu
from jax.experimental.pallas import tpu_sc as plsc

N = 100000
D = 128
K = 7
NSLOT = K - 1

BM = 1000
GRID = N // BM


def _mm_body(x_ref, w_ref, b_ref, base_ref, *t_refs):
    y = jnp.dot(
        x_ref[...],
        w_ref[...],
        preferred_element_type=jnp.float32,
    )
    base_ref[...] = y[:, :D] + b_ref[...]
    for k in range(NSLOT):
        c0 = D * (k + 1)
        t_refs[k][...] = y[:, c0:c0 + D]


def _tc_matmul(x, wcat, bias2d):
    outs = [jax.ShapeDtypeStruct((N, D), jnp.float32) for _ in range(1 + NSLOT)]
    return pl.pallas_call(
        _mm_body,
        grid=(GRID,),
        in_specs=[
            pl.BlockSpec((BM, D), lambda i: (i, 0)),
            pl.BlockSpec((D, K * D), lambda i: (0, 0)),
            pl.BlockSpec((1, D), lambda i: (0, 0)),
        ],
        out_specs=[pl.BlockSpec((BM, D), lambda i: (i, 0))] * (1 + NSLOT),
        out_shape=outs,
        compiler_params=pltpu.CompilerParams(
            dimension_semantics=("arbitrary",),
        ),
    )(x, wcat, bias2d)


NW = 32
CH = 56
WB = CH + 16
NT_FULL = 57
PER_W = CH * NT_FULL
NT_LAST = 19
TAIL = N - (NW - 1) * PER_W - (NT_LAST - 1) * CH
PAD_N = NW * PER_W


def _sc_body(t0, t1, t2, t3, t4, t5, base_hbm,
             ih0, ih1, ih2, ih3, ih4, ih5, wh0, wh1, wh2, wh3, wh4, wh5,
             out_hbm,
             i0, i1, i2, i3, i4, i5, w0s, w1s, w2s, w3s, w4s, w5s,
             g0, g1, g2, g3, g4, g5, acc_v, psem, osem):
    tabs = (t0, t1, t2, t3, t4, t5)
    idx_hbm = (ih0, ih1, ih2, ih3, ih4, ih5)
    w_hbm = (wh0, wh1, wh2, wh3, wh4, wh5)
    idxs = (i0, i1, i2, i3, i4, i5)
    wss = (w0s, w1s, w2s, w3s, w4s, w5s)
    gs = (g0, g1, g2, g3, g4, g5)
    wid = lax.axis_index("s") * 2 + lax.axis_index("c")
    w0 = pl.multiple_of(wid * PER_W, 8)
    nt = jnp.where(wid == NW - 1, NT_LAST, NT_FULL)
    is_last_w = wid == NW - 1

    for j in range(NSLOT):
        pltpu.sync_copy(idx_hbm[j].at[pl.ds(w0, PER_W)], idxs[j])

    def pf_copies(c, sbuf):
        rc = pl.multiple_of(c * CH, 8)
        bc = pl.multiple_of(wid * PER_W + c * CH, 8)
        cps = [
            pltpu.make_async_copy(
                tabs[j].at[idxs[j].at[pl.ds(rc, CH)]],
                gs[j].at[pl.ds(sbuf * CH, CH)], psem)
            for j in range(NSLOT)
        ] + [
            pltpu.make_async_copy(
                w_hbm[j].at[pl.ds(bc, CH)],
                wss[j].at[pl.ds(sbuf * WB, CH)], psem)
            for j in range(NSLOT)
        ]
        tail_cp = pltpu.make_async_copy(
            base_hbm.at[pl.ds(bc, TAIL)],
            acc_v.at[pl.ds(sbuf * CH, TAIL)], psem)
        full_cp = pltpu.make_async_copy(
            base_hbm.at[pl.ds(bc, CH)],
            acc_v.at[pl.ds(sbuf * CH, CH)], psem)
        is_tail = is_last_w & (c == NT_LAST - 1)
        return cps, tail_cp, full_cp, is_tail

    def pf_start(c, sbuf):
        cps, tail_cp, full_cp, is_tail = pf_copies(c, sbuf)
        for cp in cps:
            cp.start()
        pl.when(is_tail)(tail_cp.start)
        pl.when(jnp.logical_not(is_tail))(full_cp.start)

    def pf_wait(c, sbuf):
        cps, tail_cp, full_cp, is_tail = pf_copies(c, sbuf)
        for cp in cps:
            cp.wait()
        pl.when(is_tail)(tail_cp.wait)
        pl.when(jnp.logical_not(is_tail))(full_cp.wait)

    def out_copies(c, sbuf):
        bc = pl.multiple_of(wid * PER_W + c * CH, 8)
        tail_cp = pltpu.make_async_copy(
            acc_v.at[pl.ds(sbuf * CH, TAIL)],
            out_hbm.at[pl.ds(bc, TAIL)], osem)
        full_cp = pltpu.make_async_copy(
            acc_v.at[pl.ds(sbuf * CH, CH)],
            out_hbm.at[pl.ds(bc, CH)], osem)
        is_tail = is_last_w & (c == NT_LAST - 1)
        return tail_cp, full_cp, is_tail

    def out_start(c, sbuf):
        tail_cp, full_cp, is_tail = out_copies(c, sbuf)
        pl.when(is_tail)(tail_cp.start)
        pl.when(jnp.logical_not(is_tail))(full_cp.start)

    def out_wait(c, sbuf):
        tail_cp, full_cp, is_tail = out_copies(c, sbuf)
        pl.when(is_tail)(tail_cp.wait)
        pl.when(jnp.logical_not(is_tail))(full_cp.wait)

    pf_start(jnp.int32(0), jnp.int32(0))

    def chunk(t, carry):
        buf = t & 1
        nbuf = 1 - buf
        pl.when(t >= 1)(lambda: out_wait(t - 1, nbuf))
        pl.when(t + 1 < nt)(lambda: pf_start(t + 1, nbuf))
        pf_wait(t, buf)
        ro = buf * CH
        wo = buf * WB

        def row(i, c):
            ws = [wss[j][pl.ds(wo + i, 16)][0] for j in range(NSLOT)]
            for q in range(D // 16):
                sl = pl.ds(q * 16, 16)
                v = acc_v[ro + i, sl]
                for j in range(NSLOT):
                    v = v + ws[j] * gs[j][ro + i, sl]
                acc_v[ro + i, sl] = v
            return c

        lax.fori_loop(0, CH, row, jnp.int32(0))
        out_start(t, buf)
        return carry

    lax.fori_loop(0, nt, chunk, jnp.int32(0))
    out_wait(nt - 1, (nt - 1) & 1)


_sc_gather_accum = pl.kernel(
    _sc_body,
    out_type=jax.ShapeDtypeStruct((N, D), jnp.float32),
    mesh=plsc.VectorSubcoreMesh(core_axis_name="c", subcore_axis_name="s"),
    scratch_types=(
        [pltpu.VMEM((PER_W,), jnp.int32) for _ in range(NSLOT)]
        + [pltpu.VMEM((2 * WB,), jnp.float32) for _ in range(NSLOT)]
        + [pltpu.VMEM((2 * CH, D), jnp.float32) for _ in range(NSLOT)]
        + [
            pltpu.VMEM((2 * CH, D), jnp.float32),
            pltpu.SemaphoreType.DMA,
            pltpu.SemaphoreType.DMA,
        ]
    ),
)


def kernel(x, neighbors_indices, neighbors_weights, weight, bias):
    wcat = weight.transpose(1, 0, 2).reshape(D, K * D)
    base, *tabs = _tc_matmul(x, wcat, bias.reshape(1, D))
    pad = PAD_N - N
    idx_cols = [jnp.pad(neighbors_indices[:, j], (0, pad)) for j in range(NSLOT)]
    w_cols = [jnp.pad(neighbors_weights[:, j], (0, pad)) for j in range(NSLOT)]
    return _sc_gather_accum(*tabs, base, *idx_cols, *w_cols)
